# baseline (speedup 1.0000x reference)
import functools

import jax
import jax.numpy as jnp
from jax import lax
from jax.experimental import pallas as pl
from jax.experimental.pallas import tpu as pltpu
from jax.experimental.pallas import tpu_sc as plsc

BATCH = 4096
MAX_LEN = 200
D_MODEL = 64
LANES = 16
NUM_CORES = 2
NUM_SUBCORES = 16
NUM_WORKERS = NUM_CORES * NUM_SUBCORES
ROWS_PER_WORKER = BATCH // NUM_WORKERS
HALF = MAX_LEN // 2


def kernel(x, token_table, pos_table):
    x3 = x.astype(jnp.int32).reshape(BATCH, 2, HALF)
    mesh = plsc.VectorSubcoreMesh(core_axis_name="c", subcore_axis_name="s")

    @functools.partial(
        pl.kernel,
        out_type=jax.ShapeDtypeStruct((BATCH, MAX_LEN, D_MODEL), jnp.float32),
        mesh=mesh,
        compiler_params=pltpu.CompilerParams(use_tc_tiling_on_sc=False),
        scratch_types=[
            pltpu.VMEM((2, HALF), jnp.int32),
            pltpu.VMEM((MAX_LEN, D_MODEL), jnp.float32),
            pltpu.VMEM((MAX_LEN, D_MODEL), jnp.float32),
            pltpu.SemaphoreType.DMA,
        ],
    )
    def sc_kernel(x_hbm, tok_hbm, pos_hbm, out_hbm, idx_v, rows_v, pos_v, sem):
        wid = lax.axis_index("s") * NUM_CORES + lax.axis_index("c")
        base = wid * ROWS_PER_WORKER
        pltpu.sync_copy(pos_hbm, pos_v)

        @pl.loop(0, ROWS_PER_WORKER)
        def _(g):
            row = base + g
            pltpu.sync_copy(x_hbm.at[row], idx_v)
            c0 = pltpu.async_copy(
                tok_hbm.at[idx_v.at[0]], rows_v.at[pl.ds(0, HALF)], sem)
            c1 = pltpu.async_copy(
                tok_hbm.at[idx_v.at[1]], rows_v.at[pl.ds(HALF, HALF)], sem)
            c0.wait()
            c1.wait()

            @pl.loop(0, MAX_LEN)
            def _(l):
                for j in range(D_MODEL // LANES):
                    sl = pl.ds(j * LANES, LANES)
                    rows_v[l, sl] = rows_v[l, sl] + pos_v[l, sl]

            pltpu.sync_copy(rows_v, out_hbm.at[row])

    return sc_kernel(x3, token_table, pos_table)


# speedup vs baseline: 1.2021x; 1.2021x over previous
import functools

import jax
import jax.numpy as jnp
from jax import lax
from jax.experimental import pallas as pl
from jax.experimental.pallas import tpu as pltpu
from jax.experimental.pallas import tpu_sc as plsc

BATCH = 4096
MAX_LEN = 200
D_MODEL = 64
LANES = 16
NUM_CORES = 2
NUM_SUBCORES = 16
NUM_WORKERS = NUM_CORES * NUM_SUBCORES
BPW = BATCH // NUM_WORKERS
NBUF = 4
CHUNK_BYTES = BPW * D_MODEL * 4


def kernel(x, token_table, pos_table):
    xT = x.astype(jnp.int32).T
    pos_flat = pos_table.reshape(-1)
    mesh = plsc.VectorSubcoreMesh(core_axis_name="c", subcore_axis_name="s")

    @functools.partial(
        pl.kernel,
        out_type=jax.ShapeDtypeStruct((BATCH, MAX_LEN, D_MODEL), jnp.float32),
        mesh=mesh,
        compiler_params=pltpu.CompilerParams(use_tc_tiling_on_sc=False),
        scratch_types=[
            pltpu.VMEM((MAX_LEN, BPW), jnp.int32),
            pltpu.VMEM((MAX_LEN * D_MODEL,), jnp.float32),
            pltpu.VMEM((NBUF, BPW, D_MODEL), jnp.float32),
        ] + [pltpu.SemaphoreType.DMA] * (2 * NBUF),
    )
    def sc_kernel(xT_hbm, tok_hbm, pos_hbm, out_hbm, idx_v, pos_v, rows_v,
                  *sems):
        gsem = sems[:NBUF]
        osem = sems[NBUF:]
        wid = lax.axis_index("s") * NUM_CORES + lax.axis_index("c")
        b0 = wid * BPW
        pltpu.sync_copy(xT_hbm.at[:, pl.ds(b0, BPW)], idx_v)
        pltpu.sync_copy(pos_hbm, pos_v)

        def gather(l, k):
            pltpu.make_async_copy(
                tok_hbm.at[idx_v.at[l]], rows_v.at[k], gsem[k]).start()

        def put(l, k):
            pltpu.make_async_copy(
                rows_v.at[k], out_hbm.at[pl.ds(b0, BPW), l], osem[k]).start()

        for k in range(NBUF):
            gather(k, k)

        @pl.loop(0, MAX_LEN, step=NBUF)
        def _(g):
            for k in range(NBUF):
                l = g + k
                pltpu.make_async_copy(
                    tok_hbm.at[idx_v.at[l]], rows_v.at[k], gsem[k]).wait()
                p = [pos_v[pl.ds(l * D_MODEL + j * LANES, LANES)]
                     for j in range(D_MODEL // LANES)]

                @pl.loop(0, BPW, step=2)
                def _(i):
                    for ii in range(2):
                        for j in range(D_MODEL // LANES):
                            plsc.addupdate(
                                rows_v.at[k, i + ii, pl.ds(j * LANES, LANES)],
                                p[j])

                put(l, k)
            for k in range(NBUF):
                l = g + k
                pltpu.make_async_copy(
                    rows_v.at[k], out_hbm.at[pl.ds(b0, BPW), l],
                    osem[k]).wait()
                lnext = l + NBUF

                @pl.when(lnext < MAX_LEN)
                def _():
                    gather(lnext, k)

    return sc_kernel(xT, token_table, pos_flat)
